# Initial kernel scaffold; baseline (speedup 1.0000x reference)
import jax
import jax.numpy as jnp
from jax.experimental import pallas as pl


def kernel(x, edge_index, edge_attr, t_edge_index, t_edge_attr, lower, upper, num_graphs, params):
    raise NotImplementedError("write your pallas kernel here")


# speedup vs baseline: 1.9649x; 1.9649x over previous
import functools

import jax
import jax.numpy as jnp
from jax import lax
from jax.experimental import pallas as pl
from jax.experimental.pallas import tpu as pltpu
from jax.experimental.pallas import tpu_sc as plsc

_NC = 2
_NS = 16
_NW = _NC * _NS
_CH = 128
_ZR = 256
_TN = 512


def _tc_linear(xs_groups, weights, biases, acts, out_widths, tile_rows=_TN):
    n = xs_groups[0][0].shape[0]
    ngroups = len(xs_groups)
    nouts = len(out_widths)
    gsizes = [len(grp) for grp in xs_groups]
    has_bias = [b is not None for b in biases]

    flat_xs, in_specs = [], []
    for grp in xs_groups:
        for a in grp:
            flat_xs.append(a)
            in_specs.append(pl.BlockSpec((tile_rows, a.shape[1]), lambda i: (i, 0)))
    wlist = []
    for o in range(nouts):
        for g in range(ngroups):
            wmat = weights[o][g]
            wlist.append(wmat)
            in_specs.append(pl.BlockSpec(wmat.shape, lambda i: (0, 0)))
    blist = []
    for o in range(nouts):
        if has_bias[o]:
            blist.append(biases[o].reshape(1, -1))
            in_specs.append(pl.BlockSpec((1, out_widths[o]), lambda i: (0, 0)))
    out_specs = [pl.BlockSpec((tile_rows, w), lambda i: (i, 0)) for w in out_widths]
    out_shape = [jax.ShapeDtypeStruct((n, w), jnp.float32) for w in out_widths]

    def body(*refs):
        k = 0
        groups = []
        for g in range(ngroups):
            acc = refs[k][...]
            for j in range(1, gsizes[g]):
                acc = acc + refs[k + j][...]
            k += gsizes[g]
            groups.append(acc)
        wrefs = []
        for o in range(nouts):
            wrefs.append([refs[k + g] for g in range(ngroups)])
            k += ngroups
        brefs = []
        for o in range(nouts):
            if has_bias[o]:
                brefs.append(refs[k])
                k += 1
            else:
                brefs.append(None)
        outs = refs[k:]
        for o in range(nouts):
            acc = None
            for g in range(ngroups):
                t = lax.dot_general(
                    groups[g], wrefs[o][g][...],
                    (((1,), (1,)), ((), ())),
                    precision=lax.Precision.HIGHEST,
                    preferred_element_type=jnp.float32,
                )
                acc = t if acc is None else acc + t
            if brefs[o] is not None:
                acc = acc + brefs[o][...]
            if acts[o] == "tanh":
                acc = jnp.tanh(acc)
            outs[o][...] = acc

    return pl.pallas_call(
        body,
        grid=(n // tile_rows,),
        in_specs=in_specs,
        out_specs=out_specs,
        out_shape=out_shape,
    )(*flat_xs, *wlist, *blist)


def _sc_edge_pass(p_tab, q_tab, r_tab, dst, src):
    n, w = p_tab.shape
    e = dst.shape[0]
    epw = e // _NW
    nch = epw // _CH
    rows_per_tile = n // _NS
    nz = rows_per_tile // _ZR
    assert epw * _NW == e and nch * _CH == epw
    assert rows_per_tile * _NS == n and nz * _ZR == rows_per_tile
    assert w % 16 == 0

    mesh = plsc.VectorSubcoreMesh(core_axis_name="c", subcore_axis_name="s")

    @functools.partial(
        pl.kernel,
        mesh=mesh,
        compiler_params=pltpu.CompilerParams(use_tc_tiling_on_sc=False),
        out_type=[
            jax.ShapeDtypeStruct((n, w), jnp.float32),
            jax.ShapeDtypeStruct((n, w), jnp.float32),
        ],
        scratch_types=[
            pltpu.VMEM((_CH,), jnp.int32),
            pltpu.VMEM((_CH,), jnp.int32),
            pltpu.VMEM((_CH, w), jnp.float32),
            pltpu.VMEM((_CH, w), jnp.float32),
            pltpu.VMEM((_CH, w), jnp.float32),
            pltpu.VMEM((_ZR, w), jnp.float32),
            pltpu.VMEM_SHARED((n, w), jnp.float32),
            pltpu.SemaphoreType.DMA,
        ],
    )
    def k(p_hbm, q_hbm, r_hbm, dst_hbm, src_hbm, out0, out1, dv, sv, pv, qv, rv,
          zv, agg, sem):
        cid = lax.axis_index("c")
        sid = lax.axis_index("s")
        wid = cid * _NS + sid

        def zbody(i, c):
            for j in range(w // 16):
                zv[i, pl.ds(16 * j, 16)] = jnp.zeros((16,), jnp.float32)
            return c
        lax.fori_loop(0, _ZR, zbody, 0)
        row0 = sid * rows_per_tile
        for z in range(nz):
            pltpu.sync_copy(zv, agg.at[pl.ds(row0 + _ZR * z, _ZR)])
        plsc.subcore_barrier()

        ebase = wid * epw

        def ebody(i, c):
            off = pl.multiple_of(ebase + i * _CH, _CH)
            pltpu.sync_copy(dst_hbm.at[pl.ds(off, _CH)], dv)
            pltpu.sync_copy(src_hbm.at[pl.ds(off, _CH)], sv)
            pltpu.sync_copy(r_hbm.at[pl.ds(off, _CH)], rv)
            pltpu.async_copy(p_hbm.at[dv], pv, sem).wait()
            pltpu.async_copy(q_hbm.at[sv], qv, sem).wait()

            def rbody(r, c2):
                for j in range(w // 16):
                    s = pl.ds(16 * j, 16)
                    v = pv[r, s] + qv[r, s] + rv[r, s]
                    rv[r, s] = jnp.maximum(v, 0.01 * v)
                return c2
            lax.fori_loop(0, _CH, rbody, 0)
            pltpu.sync_copy(rv, agg.at[dv], add=True)
            return c
        lax.fori_loop(0, nch, ebody, 0)
        plsc.subcore_barrier()

        sl = pl.ds(row0, rows_per_tile)
        @pl.when(cid == 0)
        def _():
            pltpu.sync_copy(agg.at[sl], out0.at[sl])
        @pl.when(cid == 1)
        def _():
            pltpu.sync_copy(agg.at[sl], out1.at[sl])

    return k(p_tab, q_tab, r_tab, dst, src)


def _tc_final(a0, a1, u, lower, upper, tile_rows=_TN):
    n = a0.shape[0]
    w = a0.shape[1]

    def body(a0r, a1r, ur, lo, up, out):
        s = a0r[...] + a1r[...] + ur[...]
        h = jnp.tanh(s[:, 0:1])
        lov = lo[...]
        out[...] = lov + (up[...] - lov) * (h + 1.0) * 0.5

    return pl.pallas_call(
        body,
        grid=(n // tile_rows,),
        in_specs=[
            pl.BlockSpec((tile_rows, w), lambda i: (i, 0)),
            pl.BlockSpec((tile_rows, w), lambda i: (i, 0)),
            pl.BlockSpec((tile_rows, w), lambda i: (i, 0)),
            pl.BlockSpec((tile_rows, 1), lambda i: (i, 0)),
            pl.BlockSpec((tile_rows, 1), lambda i: (i, 0)),
        ],
        out_specs=pl.BlockSpec((tile_rows, 1), lambda i: (i, 0)),
        out_shape=jax.ShapeDtypeStruct((n, 1), jnp.float32),
    )(a0, a1, u, lower, upper)


def _split_lin(blk, in_c):
    wl = blk["W_lin"]
    return (wl[:, :in_c], wl[:, in_c:2 * in_c], wl[:, 2 * in_c:],
            blk["b_lin"], blk["W_up"], blk["b_up"])


def kernel(x, edge_index, edge_attr, t_edge_index, t_edge_attr, lower, upper,
           num_graphs, params):
    del num_graphs
    src1 = edge_index[0].astype(jnp.int32)
    dst1 = edge_index[1].astype(jnp.int32)
    src2 = t_edge_index[0].astype(jnp.int32)
    dst2 = t_edge_index[1].astype(jnp.int32)
    n1 = x.shape[0]
    n2 = lower.shape[0]
    g = n1 // 6

    wi1, wj1, we1, bl1, wu1, bu1 = _split_lin(params["enc1"], 6)
    wi2, wj2, we2, bl2, wu2, bu2 = _split_lin(params["enc2"], 16)
    wi3, wj3, we3, bl3, wu3, bu3 = _split_lin(params["enc3"], 32)
    wi4, wj4, we4, bl4, wu4, bu4 = _split_lin(params["dec1"], 66)
    wi5, wj5, we5, bl5, wu5, bu5 = _split_lin(params["dec2"], 32)
    wi6, wj6, we6, bl6, wu6, bu6 = _split_lin(params["dec3"], 16)

    pad16 = lambda m: jnp.pad(m, ((0, 15), (0, 0)))
    wi6p, wj6p, we6p = pad16(wi6), pad16(wj6), pad16(we6)
    bl6p = jnp.pad(bl6, (0, 15))
    wu6p = pad16(wu6)
    bu6p = jnp.pad(bu6, (0, 15))

    r1, r2, r3a, r3b = _tc_linear(
        [[edge_attr]],
        [[we1], [we2], [we3[:32]], [we3[32:]]],
        [bl1, bl2, bl3[:32], bl3[32:]],
        [None] * 4, [16, 32, 32, 32])
    r4a, r4b, r5, r6 = _tc_linear(
        [[t_edge_attr]],
        [[we4[:16]], [we4[16:]], [we5], [we6p]],
        [bl4[:16], bl4[16:], bl5, bl6p],
        [None] * 4, [16, 16, 16, 16])

    p1, q1, u1 = _tc_linear(
        [[x]], [[wi1], [wj1], [wu1]], [None, None, bu1], [None] * 3, [16, 16, 16])
    g1a, g1b = _sc_edge_pass(p1, q1, r1, dst1, src1)

    p2, q2, u2 = _tc_linear(
        [[g1a, g1b, u1]], [[wi2], [wj2], [wu2]], [None, None, bu2],
        [None] * 3, [32, 32, 32])
    g2a, g2b = _sc_edge_pass(p2, q2, r2, dst1, src1)

    p3a, p3b, q3a, q3b, u3 = _tc_linear(
        [[g2a, g2b, u2]],
        [[wi3[:32]], [wi3[32:]], [wj3[:32]], [wj3[32:]], [wu3]],
        [None, None, None, None, bu3], [None] * 5, [32, 32, 32, 32, 64])
    g3aa, g3ab = _sc_edge_pass(p3a, q3a, r3a, dst1, src1)
    g3ba, g3bb = _sc_edge_pass(p3b, q3b, r3b, dst1, src1)

    wt = params["W_t"]
    wtr = wt.reshape(wt.shape[0], 6, 64)
    wt_a = wtr[:, :, :32].reshape(wt.shape[0], 192)
    wt_b = wtr[:, :, 32:].reshape(wt.shape[0], 192)
    rg = lambda a: a.reshape(g, -1)
    (zt,) = _tc_linear(
        [[rg(g3aa), rg(g3ab)], [rg(g3ba), rg(g3bb)], [rg(u3)]],
        [[wt_a, wt_b, wt]], [params["b_t"]], ["tanh"], [wt.shape[0]])
    z2 = zt.reshape(n2, 64)

    p4a, p4b, q4a, q4b, u4 = _tc_linear(
        [[z2], [lower], [upper]],
        [[wi4[:16, :64], wi4[:16, 64:65], wi4[:16, 65:66]],
         [wi4[16:, :64], wi4[16:, 64:65], wi4[16:, 65:66]],
         [wj4[:16, :64], wj4[:16, 64:65], wj4[:16, 65:66]],
         [wj4[16:, :64], wj4[16:, 64:65], wj4[16:, 65:66]],
         [wu4[:, :64], wu4[:, 64:65], wu4[:, 65:66]]],
        [None, None, None, None, bu4], [None] * 5, [16, 16, 16, 16, 32])
    g4aa, g4ab = _sc_edge_pass(p4a, q4a, r4a, dst2, src2)
    g4ba, g4bb = _sc_edge_pass(p4b, q4b, r4b, dst2, src2)

    p5, q5, u5 = _tc_linear(
        [[g4aa, g4ab], [g4ba, g4bb], [u4]],
        [[wi5[:, :16], wi5[:, 16:32], wi5],
         [wj5[:, :16], wj5[:, 16:32], wj5],
         [wu5[:, :16], wu5[:, 16:32], wu5]],
        [None, None, bu5], [None] * 3, [16, 16, 16])
    g5a, g5b = _sc_edge_pass(p5, q5, r5, dst2, src2)

    p6, q6, u6 = _tc_linear(
        [[g5a, g5b, u5]], [[wi6p], [wj6p], [wu6p]], [None, None, bu6p],
        [None] * 3, [16, 16, 16])
    g6a, g6b = _sc_edge_pass(p6, q6, r6, dst2, src2)

    ang = _tc_final(g6a, g6b, u6, lower, upper)
    return (z2, ang)
